# baseline (speedup 1.0000x reference)
import functools

import jax
import jax.numpy as jnp
from jax import lax
from jax.experimental import pallas as pl
from jax.experimental.pallas import tpu as pltpu
from jax.experimental.pallas import tpu_sc as plsc

_L = 16


@functools.cache
def _build(batch, feat_dim, num_classes):
    info = plsc.get_sparse_core_info()
    nc, ns = info.num_cores, info.num_subcores
    nw = nc * ns
    b_per_w = batch // nw
    n_chunk = 128
    chunks = b_per_w // n_chunk
    groups = feat_dim // _L
    scale = 0.5 / batch

    mesh = plsc.VectorSubcoreMesh(core_axis_name="c", subcore_axis_name="s")

    @functools.partial(
        pl.kernel,
        out_type=jax.ShapeDtypeStruct((nw, _L), jnp.float32),
        mesh=mesh,
        compiler_params=pltpu.CompilerParams(use_tc_tiling_on_sc=False),
        scratch_types=[
            pltpu.VMEM((chunks, n_chunk), jnp.int32),
            pltpu.VMEM((b_per_w, feat_dim), jnp.float32),
            pltpu.VMEM((b_per_w, feat_dim), jnp.float32),
            pltpu.VMEM((_L,), jnp.float32),
            pltpu.SemaphoreType.DMA,
            pltpu.SemaphoreType.DMA,
        ],
    )
    def k(feat_hbm, lab_hbm, cent_hbm, out_hbm, idx_v, rows_v, feat_v, acc_v,
          gsem, fsem):
        wid = lax.axis_index("s") * nc + lax.axis_index("c")
        base = wid * b_per_w

        fcopy = pltpu.async_copy(feat_hbm.at[pl.ds(base, b_per_w)], feat_v, fsem)
        pltpu.sync_copy(lab_hbm.at[pl.ds(wid * chunks, chunks)], idx_v)
        copies = [
            pltpu.async_copy(
                cent_hbm.at[idx_v.at[j]],
                rows_v.at[pl.ds(j * n_chunk, n_chunk)],
                gsem,
            )
            for j in range(chunks)
        ]
        for c in copies:
            c.wait()
        fcopy.wait()

        def body(i, acc):
            for g in range(groups):
                f = feat_v[i, pl.ds(g * _L, _L)]
                c = rows_v[i, pl.ds(g * _L, _L)]
                d = f - c
                acc = acc + d * d
            return acc

        acc = lax.fori_loop(0, b_per_w, body, jnp.zeros((_L,), jnp.float32))
        acc_v[...] = acc * scale
        pltpu.sync_copy(acc_v, out_hbm.at[wid])

    return k


def kernel(features, labels, centers):
    batch, feat_dim = features.shape
    num_classes = centers.shape[0]
    k = _build(batch, feat_dim, num_classes)
    lab2d = labels.astype(jnp.int32).reshape(-1, 128)
    partials = k(features, lab2d, centers)
    return jnp.sum(partials)
